# baseline (speedup 1.0000x reference)
import functools

import jax
import jax.numpy as jnp
from jax import lax
from jax.experimental import pallas as pl
from jax.experimental.pallas import tpu as pltpu
from jax.experimental.pallas import tpu_sc as plsc

_NC = 2
_NS = 16
_NW = _NC * _NS
_CHUNK = 128
_BR = 1024


def _tc_degree(dst2, d):
    e = dst2.shape[0]
    bl = 2560
    grid = e // bl

    def body(dst_ref, out_ref, acc_scr):
        i = pl.program_id(0)

        @pl.when(i == 0)
        def _init():
            acc_scr[...] = jnp.zeros_like(acc_scr)

        db = dst_ref[...]
        iota = lax.broadcasted_iota(jnp.int32, (1, d), 1)
        ohu = (db % d == iota).astype(jnp.bfloat16)
        ohv = (db // d == iota).astype(jnp.bfloat16)
        dn = (((0,), (0,)), ((), ()))
        acc_scr[...] += lax.dot_general(ohv, ohu, dn,
                                        preferred_element_type=jnp.float32)

        @pl.when(i == grid - 1)
        def _fin():
            out_ref[...] = acc_scr[...]

    return pl.pallas_call(
        body,
        grid=(grid,),
        in_specs=[pl.BlockSpec((bl, 1), lambda i: (i, 0))],
        out_specs=pl.BlockSpec((d, d), lambda i: (0, 0)),
        out_shape=jax.ShapeDtypeStruct((d, d), jnp.float32),
        scratch_shapes=[pltpu.VMEM((d, d), jnp.float32)],
    )(dst2)


def _sc_edge_pass(xp, src3, dst3, zeros, npad, d, k0, k1):
    nw, k, chunk = src3.shape
    rps = npad // _NS
    mesh = plsc.VectorSubcoreMesh(core_axis_name="c", subcore_axis_name="s")

    @functools.partial(
        pl.kernel,
        mesh=mesh,
        out_type=jax.ShapeDtypeStruct((_NC, npad, d), jnp.float32),
        scratch_types=[
            pltpu.VMEM((2, chunk), jnp.int32),
            pltpu.VMEM((k, chunk), jnp.int32),
            pltpu.VMEM((2, chunk, d), jnp.float32),
            pltpu.VMEM_SHARED((npad, d), jnp.float32),
            pltpu.SemaphoreType.DMA((2,)),
            pltpu.SemaphoreType.DMA((2,)),
        ],
    )
    def edge_kernel(xp_hbm, src_hbm, dst_hbm, z_hbm, out_hbm,
                    sidx_v, dst_v, rows_v, acc_sh, sem_g, sem_s):
        cid = lax.axis_index("c")
        sid = lax.axis_index("s")
        wid = sid * _NC + cid
        kc = jnp.where(cid == 0, jnp.int32(k0), jnp.int32(k1))
        base = sid * rps
        pltpu.sync_copy(z_hbm.at[pl.ds(base, rps)], acc_sh.at[pl.ds(base, rps)])
        pltpu.sync_copy(dst_hbm.at[wid], dst_v)
        plsc.subcore_barrier()

        pltpu.sync_copy(src_hbm.at[wid, 0], sidx_v.at[0])
        pltpu.async_copy(xp_hbm.at[sidx_v.at[0]], rows_v.at[0], sem_g.at[0])
        pltpu.async_copy(src_hbm.at[wid, 1], sidx_v.at[1], sem_s.at[1])

        def body(j, c):
            p = lax.rem(j, 2)
            q = lax.rem(j + 1, 2)
            pltpu.make_async_copy(xp_hbm.at[sidx_v.at[p]], rows_v.at[p],
                                  sem_g.at[p]).wait()

            @pl.when(j + 1 < kc)
            def _next_gather():
                pltpu.make_async_copy(src_hbm.at[wid, j + 1], sidx_v.at[q],
                                      sem_s.at[q]).wait()
                pltpu.async_copy(xp_hbm.at[sidx_v.at[q]], rows_v.at[q],
                                 sem_g.at[q])

            @pl.when(j + 2 < kc)
            def _next_sidx():
                pltpu.async_copy(src_hbm.at[wid, j + 2], sidx_v.at[p],
                                 sem_s.at[p])

            pltpu.sync_copy(rows_v.at[p], acc_sh.at[dst_v.at[j]], add=True)
            return c

        lax.fori_loop(0, kc, body, 0)
        plsc.subcore_barrier()
        pltpu.sync_copy(acc_sh.at[pl.ds(base, rps)],
                        out_hbm.at[cid, pl.ds(base, rps)])

    return edge_kernel(xp, src3, dst3, zeros)


def _tc_scale(degcol, xpad, npad, d):
    grid = npad // _BR

    def body(deg_ref, x_ref, xp_ref, dinv_ref):
        dinv = 1.0 / jnp.sqrt(deg_ref[...] + 1.0)
        dinv_ref[...] = dinv
        xp_ref[...] = x_ref[...] * dinv

    return pl.pallas_call(
        body,
        grid=(grid,),
        in_specs=[
            pl.BlockSpec((_BR, 1), lambda i: (i, 0)),
            pl.BlockSpec((_BR, d), lambda i: (i, 0)),
        ],
        out_specs=[
            pl.BlockSpec((_BR, d), lambda i: (i, 0)),
            pl.BlockSpec((_BR, 1), lambda i: (i, 0)),
        ],
        out_shape=[
            jax.ShapeDtypeStruct((npad, d), jnp.float32),
            jax.ShapeDtypeStruct((npad, 1), jnp.float32),
        ],
    )(degcol, xpad)


def _elu(z):
    return jnp.where(z > 0, z, jnp.exp(z) - 1.0)


def _tc_conv_tail(acc2, xp, dinv, xres, w, b, g, be, n, npad, d):
    grid = npad // _BR

    def body(acc_ref, xp_ref, dinv_ref, x_ref, w_ref, b_ref, g_ref, be_ref,
             h_ref, xp2_ref):
        i = pl.program_id(0)
        s = acc_ref[0] + acc_ref[1] + xp_ref[...]
        agg = s * dinv_ref[...]
        pre = jnp.dot(agg, w_ref[...], preferred_element_type=jnp.float32,
                      precision=lax.Precision.HIGHEST) + b_ref[...]
        m = jnp.mean(pre, axis=-1, keepdims=True)
        v = jnp.mean((pre - m) ** 2, axis=-1, keepdims=True)
        ln = (pre - m) / jnp.sqrt(v + 1e-5) * g_ref[...] + be_ref[...]
        h = _elu(ln) + x_ref[...]
        h_ref[...] = h
        rows = i * _BR + lax.broadcasted_iota(jnp.int32, (_BR, 1), 0)
        mask = (rows < n).astype(jnp.float32)
        xp2_ref[...] = h * dinv_ref[...] * mask

    return pl.pallas_call(
        body,
        grid=(grid,),
        in_specs=[
            pl.BlockSpec((_NC, _BR, d), lambda i: (0, i, 0)),
            pl.BlockSpec((_BR, d), lambda i: (i, 0)),
            pl.BlockSpec((_BR, 1), lambda i: (i, 0)),
            pl.BlockSpec((_BR, d), lambda i: (i, 0)),
            pl.BlockSpec((d, d), lambda i: (0, 0)),
            pl.BlockSpec((1, d), lambda i: (0, 0)),
            pl.BlockSpec((1, d), lambda i: (0, 0)),
            pl.BlockSpec((1, d), lambda i: (0, 0)),
        ],
        out_specs=[
            pl.BlockSpec((_BR, d), lambda i: (i, 0)),
            pl.BlockSpec((_BR, d), lambda i: (i, 0)),
        ],
        out_shape=[
            jax.ShapeDtypeStruct((npad, d), jnp.float32),
            jax.ShapeDtypeStruct((npad, d), jnp.float32),
        ],
    )(acc2, xp, dinv, xres, w, b, g, be)


def _tc_final(acc2, xp2, dinv, hres, batch2d, w, b, g, be,
              wb, bb, wd1, bd1, wd2, bd2, n, npad, d, ngr, h2dim, p):
    grid = npad // _BR

    def body(acc_ref, xp_ref, dinv_ref, h_ref, bt_ref, w_ref, b_ref, g_ref,
             be_ref, wb_ref, bb_ref, wd1_ref, bd1_ref, wd2_ref, bd2_ref,
             bnd_ref, dom_ref, pool_scr, cnt_scr):
        i = pl.program_id(0)
        s = acc_ref[0] + acc_ref[1] + xp_ref[...]
        agg = s * dinv_ref[...]
        pre = jnp.dot(agg, w_ref[...], preferred_element_type=jnp.float32,
                      precision=lax.Precision.HIGHEST) + b_ref[...]
        m = jnp.mean(pre, axis=-1, keepdims=True)
        v = jnp.mean((pre - m) ** 2, axis=-1, keepdims=True)
        ln = (pre - m) / jnp.sqrt(v + 1e-5) * g_ref[...] + be_ref[...]
        h2 = _elu(ln) + h_ref[...]

        bnd_ref[...] = jnp.dot(h2, wb_ref[...],
                               preferred_element_type=jnp.float32,
                               precision=lax.Precision.HIGHEST) + bb_ref[...]

        @pl.when(i == 0)
        def _init():
            pool_scr[...] = jnp.zeros_like(pool_scr)
            cnt_scr[...] = jnp.zeros_like(cnt_scr)

        oh = (bt_ref[...] == lax.broadcasted_iota(jnp.int32, (1, ngr), 1)
              ).astype(jnp.float32)
        dn = (((0,), (0,)), ((), ()))
        pool_scr[...] += lax.dot_general(oh, h2, dn,
                                         preferred_element_type=jnp.float32,
                                         precision=lax.Precision.HIGHEST)
        cnt_scr[...] += lax.dot_general(oh, jnp.ones_like(h2), dn,
                                        preferred_element_type=jnp.float32,
                                        precision=lax.Precision.HIGHEST)

        @pl.when(i == grid - 1)
        def _fin():
            mean = pool_scr[...] / jnp.maximum(cnt_scr[...], 1.0)
            d1 = _elu(jnp.dot(mean, wd1_ref[...],
                              preferred_element_type=jnp.float32,
                              precision=lax.Precision.HIGHEST) + bd1_ref[...])
            dom_ref[...] = jnp.dot(d1, wd2_ref[...],
                                   preferred_element_type=jnp.float32,
                                   precision=lax.Precision.HIGHEST) + bd2_ref[...]

    return pl.pallas_call(
        body,
        grid=(grid,),
        in_specs=[
            pl.BlockSpec((_NC, _BR, d), lambda i: (0, i, 0)),
            pl.BlockSpec((_BR, d), lambda i: (i, 0)),
            pl.BlockSpec((_BR, 1), lambda i: (i, 0)),
            pl.BlockSpec((_BR, d), lambda i: (i, 0)),
            pl.BlockSpec((_BR, 1), lambda i: (i, 0)),
            pl.BlockSpec((d, d), lambda i: (0, 0)),
            pl.BlockSpec((1, d), lambda i: (0, 0)),
            pl.BlockSpec((1, d), lambda i: (0, 0)),
            pl.BlockSpec((1, d), lambda i: (0, 0)),
            pl.BlockSpec((d, 1), lambda i: (0, 0)),
            pl.BlockSpec((1, 1), lambda i: (0, 0)),
            pl.BlockSpec((d, h2dim), lambda i: (0, 0)),
            pl.BlockSpec((1, h2dim), lambda i: (0, 0)),
            pl.BlockSpec((h2dim, p), lambda i: (0, 0)),
            pl.BlockSpec((1, p), lambda i: (0, 0)),
        ],
        out_specs=[
            pl.BlockSpec((_BR, 1), lambda i: (i, 0)),
            pl.BlockSpec((ngr, p), lambda i: (0, 0)),
        ],
        out_shape=[
            jax.ShapeDtypeStruct((npad, 1), jnp.float32),
            jax.ShapeDtypeStruct((ngr, p), jnp.float32),
        ],
        scratch_shapes=[
            pltpu.VMEM((ngr, d), jnp.float32),
            pltpu.VMEM((ngr, d), jnp.float32),
        ],
    )(acc2, xp2, dinv, hres, batch2d, w, b, g, be,
      wb, bb, wd1, bd1, wd2, bd2)


def kernel(x, edge_index, batch, Wc1, bc1, g1, be1, Wc2, bc2, g2, be2,
           Wb, bb, Wd1, bd1, Wd2, bd2):
    n, d = x.shape
    e = edge_index.shape[1]
    ngr = 16
    h2dim = Wd1.shape[1]
    p = Wd2.shape[1]

    npad = -(-n // 2048) * 2048
    dummy = npad - 1

    f0 = 0.345
    c_total = -(-e // _CHUNK)
    k0 = max(2, -(-int(c_total * f0) // _NS))
    k1 = max(2, -(-(c_total - _NS * k0) // _NS))
    kmax = max(k0, k1)
    ntot = _NS * (k0 + k1) * _CHUNK

    def _layout(idx):
        flat = jnp.concatenate(
            [idx, jnp.full((ntot - e,), dummy, jnp.int32)])
        c0 = flat[:_NS * k0 * _CHUNK].reshape(_NS, k0, _CHUNK)
        c1 = flat[_NS * k0 * _CHUNK:].reshape(_NS, k1, _CHUNK)
        c0 = jnp.pad(c0, ((0, 0), (0, kmax - k0), (0, 0)),
                     constant_values=dummy)
        c1 = jnp.pad(c1, ((0, 0), (0, kmax - k1), (0, 0)),
                     constant_values=dummy)
        return jnp.stack([c0, c1], axis=1).reshape(_NW, kmax, _CHUNK)

    src = edge_index[0].astype(jnp.int32)
    dst = edge_index[1].astype(jnp.int32)
    src3 = _layout(src)
    dst3 = _layout(dst)

    xpad = jnp.pad(x, ((0, npad - n), (0, 0)))
    batch2d = jnp.pad(batch.astype(jnp.int32), (0, npad - n),
                      constant_values=ngr).reshape(npad, 1)

    zeros_acc = jnp.zeros((npad, d), jnp.float32)

    ebl = 2560
    epad2 = -(-e // ebl) * ebl
    dst2 = jnp.concatenate(
        [dst, jnp.full((epad2 - e,), dummy, jnp.int32)]).reshape(epad2, 1)
    degmat = _tc_degree(dst2, d)
    degcol = degmat.reshape(d * d, 1)[:npad]
    xp1, dinv = _tc_scale(degcol, xpad, npad, d)

    acc1 = _sc_edge_pass(xp1, src3, dst3, zeros_acc, npad, d, k0, k1)
    h, xp2 = _tc_conv_tail(acc1, xp1, dinv, xpad, Wc1,
                           bc1.reshape(1, d), g1.reshape(1, d),
                           be1.reshape(1, d), n, npad, d)

    acc2 = _sc_edge_pass(xp2, src3, dst3, zeros_acc, npad, d, k0, k1)
    bnd, dom = _tc_final(acc2, xp2, dinv, h, batch2d, Wc2,
                         bc2.reshape(1, d), g2.reshape(1, d),
                         be2.reshape(1, d), Wb, bb.reshape(1, 1),
                         Wd1, bd1.reshape(1, h2dim), Wd2, bd2.reshape(1, p),
                         n, npad, d, ngr, h2dim, p)

    return bnd[:n, 0], dom


# speedup vs baseline: 1.1237x; 1.1237x over previous
import functools

import jax
import jax.numpy as jnp
from jax import lax
from jax.experimental import pallas as pl
from jax.experimental.pallas import tpu as pltpu
from jax.experimental.pallas import tpu_sc as plsc

_NC = 2
_NS = 16
_NW = _NC * _NS
_CHUNK = 128
_BR = 1024


def _tc_degree(dst2, d):
    e = dst2.shape[0]
    bl = 2560
    grid = e // bl

    def body(dst_ref, out_ref, acc_scr):
        i = pl.program_id(0)

        @pl.when(i == 0)
        def _init():
            acc_scr[...] = jnp.zeros_like(acc_scr)

        db = dst_ref[...]
        iota = lax.broadcasted_iota(jnp.int32, (1, d), 1)
        ohu = (db % d == iota).astype(jnp.bfloat16)
        ohv = (db // d == iota).astype(jnp.bfloat16)
        dn = (((0,), (0,)), ((), ()))
        acc_scr[...] += lax.dot_general(ohv, ohu, dn,
                                        preferred_element_type=jnp.float32)

        @pl.when(i == grid - 1)
        def _fin():
            out_ref[...] = acc_scr[...]

    return pl.pallas_call(
        body,
        grid=(grid,),
        in_specs=[pl.BlockSpec((bl, 1), lambda i: (i, 0))],
        out_specs=pl.BlockSpec((d, d), lambda i: (0, 0)),
        out_shape=jax.ShapeDtypeStruct((d, d), jnp.float32),
        scratch_shapes=[pltpu.VMEM((d, d), jnp.float32)],
    )(dst2)


def _sc_edge_pass(xp, src3, dst3, zeros, npad, d, k0, k1):
    nw, k, chunk = src3.shape
    rps = npad // _NS
    mesh = plsc.VectorSubcoreMesh(core_axis_name="c", subcore_axis_name="s")

    @functools.partial(
        pl.kernel,
        mesh=mesh,
        out_type=jax.ShapeDtypeStruct((_NC, npad, d), jnp.float32),
        scratch_types=[
            pltpu.VMEM((2, chunk), jnp.int32),
            pltpu.VMEM((k, chunk), jnp.int32),
            pltpu.VMEM((2, chunk, d), jnp.float32),
            pltpu.VMEM_SHARED((npad, d), jnp.float32),
            pltpu.SemaphoreType.DMA((2,)),
            pltpu.SemaphoreType.DMA((2,)),
        ],
    )
    def edge_kernel(xp_hbm, src_hbm, dst_hbm, z_hbm, out_hbm,
                    sidx_v, dst_v, rows_v, acc_sh, sem_g, sem_s):
        cid = lax.axis_index("c")
        sid = lax.axis_index("s")
        wid = sid * _NC + cid
        kc = jnp.where(cid == 0, jnp.int32(k0), jnp.int32(k1))
        base = sid * rps
        pltpu.sync_copy(z_hbm.at[pl.ds(base, rps)], acc_sh.at[pl.ds(base, rps)])
        pltpu.sync_copy(dst_hbm.at[wid], dst_v)
        plsc.subcore_barrier()

        pltpu.sync_copy(src_hbm.at[wid, 0], sidx_v.at[0])
        pltpu.async_copy(xp_hbm.at[sidx_v.at[0]], rows_v.at[0], sem_g.at[0])
        pltpu.async_copy(src_hbm.at[wid, 1], sidx_v.at[1], sem_s.at[1])

        def body(j, c):
            p = lax.rem(j, 2)
            q = lax.rem(j + 1, 2)
            pltpu.make_async_copy(xp_hbm.at[sidx_v.at[p]], rows_v.at[p],
                                  sem_g.at[p]).wait()

            @pl.when(j + 1 < kc)
            def _next_gather():
                pltpu.make_async_copy(src_hbm.at[wid, j + 1], sidx_v.at[q],
                                      sem_s.at[q]).wait()
                pltpu.async_copy(xp_hbm.at[sidx_v.at[q]], rows_v.at[q],
                                 sem_g.at[q])

            @pl.when(j + 2 < kc)
            def _next_sidx():
                pltpu.async_copy(src_hbm.at[wid, j + 2], sidx_v.at[p],
                                 sem_s.at[p])

            pltpu.sync_copy(rows_v.at[p], acc_sh.at[dst_v.at[j]], add=True)
            return c

        lax.fori_loop(0, kc, body, 0)
        plsc.subcore_barrier()
        pltpu.sync_copy(acc_sh.at[pl.ds(base, rps)],
                        out_hbm.at[cid, pl.ds(base, rps)])

    return edge_kernel(xp, src3, dst3, zeros)


def _tc_scale(degcol, xpad, npad, d):
    grid = npad // _BR

    def body(deg_ref, x_ref, xp_ref, dinv_ref):
        dinv = 1.0 / jnp.sqrt(deg_ref[...] + 1.0)
        dinv_ref[...] = dinv
        xp_ref[...] = x_ref[...] * dinv

    return pl.pallas_call(
        body,
        grid=(grid,),
        in_specs=[
            pl.BlockSpec((_BR, 1), lambda i: (i, 0)),
            pl.BlockSpec((_BR, d), lambda i: (i, 0)),
        ],
        out_specs=[
            pl.BlockSpec((_BR, d), lambda i: (i, 0)),
            pl.BlockSpec((_BR, 1), lambda i: (i, 0)),
        ],
        out_shape=[
            jax.ShapeDtypeStruct((npad, d), jnp.float32),
            jax.ShapeDtypeStruct((npad, 1), jnp.float32),
        ],
    )(degcol, xpad)


def _elu(z):
    return jnp.where(z > 0, z, jnp.exp(z) - 1.0)


def _tc_conv_tail(acc2, xp, dinv, xres, w, b, g, be, n, npad, d):
    grid = npad // _BR

    def body(acc_ref, xp_ref, dinv_ref, x_ref, w_ref, b_ref, g_ref, be_ref,
             h_ref, xp2_ref):
        i = pl.program_id(0)
        s = acc_ref[0] + acc_ref[1] + xp_ref[...]
        agg = s * dinv_ref[...]
        pre = jnp.dot(agg, w_ref[...], preferred_element_type=jnp.float32,
                      precision=lax.Precision.HIGHEST) + b_ref[...]
        m = jnp.mean(pre, axis=-1, keepdims=True)
        v = jnp.mean((pre - m) ** 2, axis=-1, keepdims=True)
        ln = (pre - m) / jnp.sqrt(v + 1e-5) * g_ref[...] + be_ref[...]
        h = _elu(ln) + x_ref[...]
        h_ref[...] = h
        rows = i * _BR + lax.broadcasted_iota(jnp.int32, (_BR, 1), 0)
        mask = (rows < n).astype(jnp.float32)
        xp2_ref[...] = h * dinv_ref[...] * mask

    return pl.pallas_call(
        body,
        grid=(grid,),
        in_specs=[
            pl.BlockSpec((_NC, _BR, d), lambda i: (0, i, 0)),
            pl.BlockSpec((_BR, d), lambda i: (i, 0)),
            pl.BlockSpec((_BR, 1), lambda i: (i, 0)),
            pl.BlockSpec((_BR, d), lambda i: (i, 0)),
            pl.BlockSpec((d, d), lambda i: (0, 0)),
            pl.BlockSpec((1, d), lambda i: (0, 0)),
            pl.BlockSpec((1, d), lambda i: (0, 0)),
            pl.BlockSpec((1, d), lambda i: (0, 0)),
        ],
        out_specs=[
            pl.BlockSpec((_BR, d), lambda i: (i, 0)),
            pl.BlockSpec((_BR, d), lambda i: (i, 0)),
        ],
        out_shape=[
            jax.ShapeDtypeStruct((npad, d), jnp.float32),
            jax.ShapeDtypeStruct((npad, d), jnp.float32),
        ],
    )(acc2, xp, dinv, xres, w, b, g, be)


def _tc_final(acc2, xp2, dinv, hres, batch2d, w, b, g, be,
              wb, bb, wd1, bd1, wd2, bd2, n, npad, d, ngr, h2dim, p):
    grid = npad // _BR

    def body(acc_ref, xp_ref, dinv_ref, h_ref, bt_ref, w_ref, b_ref, g_ref,
             be_ref, wb_ref, bb_ref, wd1_ref, bd1_ref, wd2_ref, bd2_ref,
             bnd_ref, dom_ref, pool_scr, cnt_scr):
        i = pl.program_id(0)
        s = acc_ref[0] + acc_ref[1] + xp_ref[...]
        agg = s * dinv_ref[...]
        pre = jnp.dot(agg, w_ref[...], preferred_element_type=jnp.float32,
                      precision=lax.Precision.HIGHEST) + b_ref[...]
        m = jnp.mean(pre, axis=-1, keepdims=True)
        v = jnp.mean((pre - m) ** 2, axis=-1, keepdims=True)
        ln = (pre - m) / jnp.sqrt(v + 1e-5) * g_ref[...] + be_ref[...]
        h2 = _elu(ln) + h_ref[...]

        bnd_ref[...] = jnp.dot(h2, wb_ref[...],
                               preferred_element_type=jnp.float32,
                               precision=lax.Precision.HIGHEST) + bb_ref[...]

        @pl.when(i == 0)
        def _init():
            pool_scr[...] = jnp.zeros_like(pool_scr)
            cnt_scr[...] = jnp.zeros_like(cnt_scr)

        oh = (bt_ref[...] == lax.broadcasted_iota(jnp.int32, (1, ngr), 1)
              ).astype(jnp.float32)
        dn = (((0,), (0,)), ((), ()))
        pool_scr[...] += lax.dot_general(oh, h2, dn,
                                         preferred_element_type=jnp.float32,
                                         precision=lax.Precision.HIGHEST)
        cnt_scr[...] += lax.dot_general(oh, jnp.ones_like(h2), dn,
                                        preferred_element_type=jnp.float32,
                                        precision=lax.Precision.HIGHEST)

        @pl.when(i == grid - 1)
        def _fin():
            mean = pool_scr[...] / jnp.maximum(cnt_scr[...], 1.0)
            d1 = _elu(jnp.dot(mean, wd1_ref[...],
                              preferred_element_type=jnp.float32,
                              precision=lax.Precision.HIGHEST) + bd1_ref[...])
            dom_ref[...] = jnp.dot(d1, wd2_ref[...],
                                   preferred_element_type=jnp.float32,
                                   precision=lax.Precision.HIGHEST) + bd2_ref[...]

    return pl.pallas_call(
        body,
        grid=(grid,),
        in_specs=[
            pl.BlockSpec((_NC, _BR, d), lambda i: (0, i, 0)),
            pl.BlockSpec((_BR, d), lambda i: (i, 0)),
            pl.BlockSpec((_BR, 1), lambda i: (i, 0)),
            pl.BlockSpec((_BR, d), lambda i: (i, 0)),
            pl.BlockSpec((_BR, 1), lambda i: (i, 0)),
            pl.BlockSpec((d, d), lambda i: (0, 0)),
            pl.BlockSpec((1, d), lambda i: (0, 0)),
            pl.BlockSpec((1, d), lambda i: (0, 0)),
            pl.BlockSpec((1, d), lambda i: (0, 0)),
            pl.BlockSpec((d, 1), lambda i: (0, 0)),
            pl.BlockSpec((1, 1), lambda i: (0, 0)),
            pl.BlockSpec((d, h2dim), lambda i: (0, 0)),
            pl.BlockSpec((1, h2dim), lambda i: (0, 0)),
            pl.BlockSpec((h2dim, p), lambda i: (0, 0)),
            pl.BlockSpec((1, p), lambda i: (0, 0)),
        ],
        out_specs=[
            pl.BlockSpec((_BR, 1), lambda i: (i, 0)),
            pl.BlockSpec((ngr, p), lambda i: (0, 0)),
        ],
        out_shape=[
            jax.ShapeDtypeStruct((npad, 1), jnp.float32),
            jax.ShapeDtypeStruct((ngr, p), jnp.float32),
        ],
        scratch_shapes=[
            pltpu.VMEM((ngr, d), jnp.float32),
            pltpu.VMEM((ngr, d), jnp.float32),
        ],
    )(acc2, xp2, dinv, hres, batch2d, w, b, g, be,
      wb, bb, wd1, bd1, wd2, bd2)


def kernel(x, edge_index, batch, Wc1, bc1, g1, be1, Wc2, bc2, g2, be2,
           Wb, bb, Wd1, bd1, Wd2, bd2):
    n, d = x.shape
    e = edge_index.shape[1]
    ngr = 16
    h2dim = Wd1.shape[1]
    p = Wd2.shape[1]

    npad = -(-n // 2048) * 2048
    dummy = npad - 1

    f0 = 0.655
    c_total = -(-e // _CHUNK)
    k0 = max(2, -(-int(c_total * f0) // _NS))
    k1 = max(2, -(-(c_total - _NS * k0) // _NS))
    kmax = max(k0, k1)
    ntot = _NS * (k0 + k1) * _CHUNK

    def _layout(idx):
        flat = jnp.concatenate(
            [idx, jnp.full((ntot - e,), dummy, jnp.int32)])
        c0 = flat[:_NS * k0 * _CHUNK].reshape(_NS, k0, _CHUNK)
        c1 = flat[_NS * k0 * _CHUNK:].reshape(_NS, k1, _CHUNK)
        c0 = jnp.pad(c0, ((0, 0), (0, kmax - k0), (0, 0)),
                     constant_values=dummy)
        c1 = jnp.pad(c1, ((0, 0), (0, kmax - k1), (0, 0)),
                     constant_values=dummy)
        return jnp.stack([c0, c1], axis=1).reshape(_NW, kmax, _CHUNK)

    src = edge_index[0].astype(jnp.int32)
    dst = edge_index[1].astype(jnp.int32)
    src3 = _layout(src)
    dst3 = _layout(dst)

    xpad = jnp.pad(x, ((0, npad - n), (0, 0)))
    batch2d = jnp.pad(batch.astype(jnp.int32), (0, npad - n),
                      constant_values=ngr).reshape(npad, 1)

    zeros_acc = jnp.zeros((npad, d), jnp.float32)

    ebl = 2560
    epad2 = -(-e // ebl) * ebl
    dst2 = jnp.concatenate(
        [dst, jnp.full((epad2 - e,), dummy, jnp.int32)]).reshape(epad2, 1)
    degmat = _tc_degree(dst2, d)
    degcol = degmat.reshape(d * d, 1)[:npad]
    xp1, dinv = _tc_scale(degcol, xpad, npad, d)

    acc1 = _sc_edge_pass(xp1, src3, dst3, zeros_acc, npad, d, k0, k1)
    h, xp2 = _tc_conv_tail(acc1, xp1, dinv, xpad, Wc1,
                           bc1.reshape(1, d), g1.reshape(1, d),
                           be1.reshape(1, d), n, npad, d)

    acc2 = _sc_edge_pass(xp2, src3, dst3, zeros_acc, npad, d, k0, k1)
    bnd, dom = _tc_final(acc2, xp2, dinv, h, batch2d, Wc2,
                         bc2.reshape(1, d), g2.reshape(1, d),
                         be2.reshape(1, d), Wb, bb.reshape(1, 1),
                         Wd1, bd1.reshape(1, h2dim), Wd2, bd2.reshape(1, p),
                         n, npad, d, ngr, h2dim, p)

    return bnd[:n, 0], dom


# speedup vs baseline: 1.2189x; 1.0847x over previous
import functools

import jax
import jax.numpy as jnp
from jax import lax
from jax.experimental import pallas as pl
from jax.experimental.pallas import tpu as pltpu
from jax.experimental.pallas import tpu_sc as plsc

_NC = 2
_NS = 16
_NW = _NC * _NS
_CHUNK = 128
_BR = 1024


def _tc_degree(dst_col, dst_row, d):
    e = dst_col.shape[0]
    bl = 2560
    grid = e // bl
    shift = d.bit_length() - 1

    def body(dstc_ref, dstr_ref, out_ref, acc_scr):
        i = pl.program_id(0)

        @pl.when(i == 0)
        def _init():
            acc_scr[...] = jnp.zeros_like(acc_scr)

        dbc = dstc_ref[...]
        dbr = dstr_ref[...]
        sub_iota = lax.broadcasted_iota(jnp.int32, (d, 1), 0)
        lane_iota = lax.broadcasted_iota(jnp.int32, (1, d), 1)
        ohu_t = ((dbr & (d - 1)) == sub_iota).astype(jnp.bfloat16)
        ohv = (lax.shift_right_logical(dbc, shift) == lane_iota
               ).astype(jnp.bfloat16)
        dn = (((1,), (0,)), ((), ()))
        acc_scr[...] += lax.dot_general(ohu_t, ohv, dn,
                                        preferred_element_type=jnp.float32)

        @pl.when(i == grid - 1)
        def _fin():
            out_ref[...] = acc_scr[...]

    return pl.pallas_call(
        body,
        grid=(grid,),
        in_specs=[
            pl.BlockSpec((bl, 1), lambda i: (i, 0)),
            pl.BlockSpec((1, bl), lambda i: (0, i)),
        ],
        out_specs=pl.BlockSpec((d, d), lambda i: (0, 0)),
        out_shape=jax.ShapeDtypeStruct((d, d), jnp.float32),
        scratch_shapes=[pltpu.VMEM((d, d), jnp.float32)],
    )(dst_col, dst_row)


def _sc_edge_pass(xp, src3, dst3, npad, d, k0, k1):
    nw, k, chunk = src3.shape
    rps = npad // _NS
    mesh = plsc.VectorSubcoreMesh(core_axis_name="c", subcore_axis_name="s")

    @functools.partial(
        pl.kernel,
        mesh=mesh,
        out_type=jax.ShapeDtypeStruct((_NC, npad, d), jnp.float32),
        scratch_types=[
            pltpu.VMEM((2, chunk), jnp.int32),
            pltpu.VMEM((k, chunk), jnp.int32),
            pltpu.VMEM((2, chunk, d), jnp.float32),
            pltpu.VMEM_SHARED((npad, d), jnp.float32),
            pltpu.SemaphoreType.DMA((2,)),
            pltpu.SemaphoreType.DMA((2,)),
        ],
    )
    def edge_kernel(xp_hbm, src_hbm, dst_hbm, out_hbm,
                    sidx_v, dst_v, rows_v, acc_sh, sem_g, sem_s):
        cid = lax.axis_index("c")
        sid = lax.axis_index("s")
        wid = sid * _NC + cid
        kc = jnp.where(cid == 0, jnp.int32(k0), jnp.int32(k1))
        base = sid * rps

        z16 = jnp.zeros((16,), jnp.float32)

        def zrow(r, c):
            for l in range(d // 16):
                rows_v[0, r, pl.ds(l * 16, 16)] = z16
            return c

        lax.fori_loop(0, chunk, zrow, 0)

        def zcp(t, c):
            pltpu.sync_copy(rows_v.at[0],
                            acc_sh.at[pl.ds(base + t * chunk, chunk)])
            return c

        lax.fori_loop(0, rps // chunk, zcp, 0)
        pltpu.sync_copy(dst_hbm.at[wid], dst_v)
        plsc.subcore_barrier()

        pltpu.sync_copy(src_hbm.at[wid, 0], sidx_v.at[0])
        pltpu.async_copy(xp_hbm.at[sidx_v.at[0]], rows_v.at[0], sem_g.at[0])
        pltpu.async_copy(src_hbm.at[wid, 1], sidx_v.at[1], sem_s.at[1])

        def body(j, c):
            p = lax.rem(j, 2)
            q = lax.rem(j + 1, 2)
            pltpu.make_async_copy(xp_hbm.at[sidx_v.at[p]], rows_v.at[p],
                                  sem_g.at[p]).wait()

            @pl.when(j + 1 < kc)
            def _next_gather():
                pltpu.make_async_copy(src_hbm.at[wid, j + 1], sidx_v.at[q],
                                      sem_s.at[q]).wait()
                pltpu.async_copy(xp_hbm.at[sidx_v.at[q]], rows_v.at[q],
                                 sem_g.at[q])

            @pl.when(j + 2 < kc)
            def _next_sidx():
                pltpu.async_copy(src_hbm.at[wid, j + 2], sidx_v.at[p],
                                 sem_s.at[p])

            pltpu.sync_copy(rows_v.at[p], acc_sh.at[dst_v.at[j]], add=True)
            return c

        lax.fori_loop(0, kc, body, 0)
        plsc.subcore_barrier()
        pltpu.sync_copy(acc_sh.at[pl.ds(base, rps)],
                        out_hbm.at[cid, pl.ds(base, rps)])

    return edge_kernel(xp, src3, dst3)


def _tc_scale(degcol, xpad, npad, d):
    grid = npad // _BR

    def body(deg_ref, x_ref, xp_ref, dinv_ref):
        dinv = 1.0 / jnp.sqrt(deg_ref[...] + 1.0)
        dinv_ref[...] = dinv
        xp_ref[...] = x_ref[...] * dinv

    return pl.pallas_call(
        body,
        grid=(grid,),
        in_specs=[
            pl.BlockSpec((_BR, 1), lambda i: (i, 0)),
            pl.BlockSpec((_BR, d), lambda i: (i, 0)),
        ],
        out_specs=[
            pl.BlockSpec((_BR, d), lambda i: (i, 0)),
            pl.BlockSpec((_BR, 1), lambda i: (i, 0)),
        ],
        out_shape=[
            jax.ShapeDtypeStruct((npad, d), jnp.float32),
            jax.ShapeDtypeStruct((npad, 1), jnp.float32),
        ],
    )(degcol, xpad)


def _elu(z):
    return jnp.where(z > 0, z, jnp.exp(z) - 1.0)


def _tc_conv_tail(acc2, xp, dinv, xres, w, b, g, be, n, npad, d):
    grid = npad // _BR

    def body(acc_ref, xp_ref, dinv_ref, x_ref, w_ref, b_ref, g_ref, be_ref,
             h_ref, xp2_ref):
        i = pl.program_id(0)
        s = acc_ref[0] + acc_ref[1] + xp_ref[...]
        agg = s * dinv_ref[...]
        pre = jnp.dot(agg, w_ref[...], preferred_element_type=jnp.float32,
                      precision=lax.Precision.HIGHEST) + b_ref[...]
        m = jnp.mean(pre, axis=-1, keepdims=True)
        v = jnp.mean((pre - m) ** 2, axis=-1, keepdims=True)
        ln = (pre - m) / jnp.sqrt(v + 1e-5) * g_ref[...] + be_ref[...]
        h = _elu(ln) + x_ref[...]
        h_ref[...] = h
        rows = i * _BR + lax.broadcasted_iota(jnp.int32, (_BR, 1), 0)
        mask = (rows < n).astype(jnp.float32)
        xp2_ref[...] = h * dinv_ref[...] * mask

    return pl.pallas_call(
        body,
        grid=(grid,),
        in_specs=[
            pl.BlockSpec((_NC, _BR, d), lambda i: (0, i, 0)),
            pl.BlockSpec((_BR, d), lambda i: (i, 0)),
            pl.BlockSpec((_BR, 1), lambda i: (i, 0)),
            pl.BlockSpec((_BR, d), lambda i: (i, 0)),
            pl.BlockSpec((d, d), lambda i: (0, 0)),
            pl.BlockSpec((1, d), lambda i: (0, 0)),
            pl.BlockSpec((1, d), lambda i: (0, 0)),
            pl.BlockSpec((1, d), lambda i: (0, 0)),
        ],
        out_specs=[
            pl.BlockSpec((_BR, d), lambda i: (i, 0)),
            pl.BlockSpec((_BR, d), lambda i: (i, 0)),
        ],
        out_shape=[
            jax.ShapeDtypeStruct((npad, d), jnp.float32),
            jax.ShapeDtypeStruct((npad, d), jnp.float32),
        ],
    )(acc2, xp, dinv, xres, w, b, g, be)


def _tc_final(acc2, xp2, dinv, hres, batch2d, w, b, g, be,
              wb, bb, wd1, bd1, wd2, bd2, n, npad, d, ngr, h2dim, p):
    grid = npad // _BR

    def body(acc_ref, xp_ref, dinv_ref, h_ref, bt_ref, w_ref, b_ref, g_ref,
             be_ref, wb_ref, bb_ref, wd1_ref, bd1_ref, wd2_ref, bd2_ref,
             bnd_ref, dom_ref, pool_scr, cnt_scr):
        i = pl.program_id(0)
        s = acc_ref[0] + acc_ref[1] + xp_ref[...]
        agg = s * dinv_ref[...]
        pre = jnp.dot(agg, w_ref[...], preferred_element_type=jnp.float32,
                      precision=lax.Precision.HIGHEST) + b_ref[...]
        m = jnp.mean(pre, axis=-1, keepdims=True)
        v = jnp.mean((pre - m) ** 2, axis=-1, keepdims=True)
        ln = (pre - m) / jnp.sqrt(v + 1e-5) * g_ref[...] + be_ref[...]
        h2 = _elu(ln) + h_ref[...]

        bnd_ref[...] = jnp.dot(h2, wb_ref[...],
                               preferred_element_type=jnp.float32,
                               precision=lax.Precision.HIGHEST) + bb_ref[...]

        @pl.when(i == 0)
        def _init():
            pool_scr[...] = jnp.zeros_like(pool_scr)
            cnt_scr[...] = jnp.zeros_like(cnt_scr)

        oh = (bt_ref[...] == lax.broadcasted_iota(jnp.int32, (1, ngr), 1)
              ).astype(jnp.float32)
        dn = (((0,), (0,)), ((), ()))
        pool_scr[...] += lax.dot_general(oh, h2, dn,
                                         preferred_element_type=jnp.float32,
                                         precision=lax.Precision.HIGHEST)
        cnt_scr[...] += lax.dot_general(oh, jnp.ones_like(h2), dn,
                                        preferred_element_type=jnp.float32,
                                        precision=lax.Precision.HIGHEST)

        @pl.when(i == grid - 1)
        def _fin():
            mean = pool_scr[...] / jnp.maximum(cnt_scr[...], 1.0)
            d1 = _elu(jnp.dot(mean, wd1_ref[...],
                              preferred_element_type=jnp.float32,
                              precision=lax.Precision.HIGHEST) + bd1_ref[...])
            dom_ref[...] = jnp.dot(d1, wd2_ref[...],
                                   preferred_element_type=jnp.float32,
                                   precision=lax.Precision.HIGHEST) + bd2_ref[...]

    return pl.pallas_call(
        body,
        grid=(grid,),
        in_specs=[
            pl.BlockSpec((_NC, _BR, d), lambda i: (0, i, 0)),
            pl.BlockSpec((_BR, d), lambda i: (i, 0)),
            pl.BlockSpec((_BR, 1), lambda i: (i, 0)),
            pl.BlockSpec((_BR, d), lambda i: (i, 0)),
            pl.BlockSpec((_BR, 1), lambda i: (i, 0)),
            pl.BlockSpec((d, d), lambda i: (0, 0)),
            pl.BlockSpec((1, d), lambda i: (0, 0)),
            pl.BlockSpec((1, d), lambda i: (0, 0)),
            pl.BlockSpec((1, d), lambda i: (0, 0)),
            pl.BlockSpec((d, 1), lambda i: (0, 0)),
            pl.BlockSpec((1, 1), lambda i: (0, 0)),
            pl.BlockSpec((d, h2dim), lambda i: (0, 0)),
            pl.BlockSpec((1, h2dim), lambda i: (0, 0)),
            pl.BlockSpec((h2dim, p), lambda i: (0, 0)),
            pl.BlockSpec((1, p), lambda i: (0, 0)),
        ],
        out_specs=[
            pl.BlockSpec((_BR, 1), lambda i: (i, 0)),
            pl.BlockSpec((ngr, p), lambda i: (0, 0)),
        ],
        out_shape=[
            jax.ShapeDtypeStruct((npad, 1), jnp.float32),
            jax.ShapeDtypeStruct((ngr, p), jnp.float32),
        ],
        scratch_shapes=[
            pltpu.VMEM((ngr, d), jnp.float32),
            pltpu.VMEM((ngr, d), jnp.float32),
        ],
    )(acc2, xp2, dinv, hres, batch2d, w, b, g, be,
      wb, bb, wd1, bd1, wd2, bd2)


def kernel(x, edge_index, batch, Wc1, bc1, g1, be1, Wc2, bc2, g2, be2,
           Wb, bb, Wd1, bd1, Wd2, bd2):
    n, d = x.shape
    e = edge_index.shape[1]
    ngr = 16
    h2dim = Wd1.shape[1]
    p = Wd2.shape[1]

    npad = -(-n // 2048) * 2048
    dummy = npad - 1

    f0 = 0.63
    c_total = -(-e // _CHUNK)
    k0 = max(2, -(-int(c_total * f0) // _NS))
    k1 = max(2, -(-(c_total - _NS * k0) // _NS))
    kmax = max(k0, k1)
    ntot = _NS * (k0 + k1) * _CHUNK

    def _layout(idx):
        flat = jnp.concatenate(
            [idx, jnp.full((ntot - e,), dummy, jnp.int32)])
        c0 = flat[:_NS * k0 * _CHUNK].reshape(_NS, k0, _CHUNK)
        c1 = flat[_NS * k0 * _CHUNK:].reshape(_NS, k1, _CHUNK)
        c0 = jnp.pad(c0, ((0, 0), (0, kmax - k0), (0, 0)),
                     constant_values=dummy)
        c1 = jnp.pad(c1, ((0, 0), (0, kmax - k1), (0, 0)),
                     constant_values=dummy)
        return jnp.stack([c0, c1], axis=1).reshape(_NW, kmax, _CHUNK)

    src = edge_index[0].astype(jnp.int32)
    dst = edge_index[1].astype(jnp.int32)
    src3 = _layout(src)
    dst3 = _layout(dst)

    xpad = jnp.pad(x, ((0, npad - n), (0, 0)))
    batch2d = jnp.pad(batch.astype(jnp.int32), (0, npad - n),
                      constant_values=ngr).reshape(npad, 1)

    ebl = 2560
    epad2 = -(-e // ebl) * ebl
    dflat = jnp.concatenate(
        [dst, jnp.full((epad2 - e,), dummy, jnp.int32)])
    degmat = _tc_degree(dflat.reshape(epad2, 1), dflat.reshape(1, epad2), d)
    degcol = degmat.T.reshape(d * d, 1)[:npad]
    xp1, dinv = _tc_scale(degcol, xpad, npad, d)

    acc1 = _sc_edge_pass(xp1, src3, dst3, npad, d, k0, k1)
    h, xp2 = _tc_conv_tail(acc1, xp1, dinv, xpad, Wc1,
                           bc1.reshape(1, d), g1.reshape(1, d),
                           be1.reshape(1, d), n, npad, d)

    acc2 = _sc_edge_pass(xp2, src3, dst3, npad, d, k0, k1)
    bnd, dom = _tc_final(acc2, xp2, dinv, h, batch2d, Wc2,
                         bc2.reshape(1, d), g2.reshape(1, d),
                         be2.reshape(1, d), Wb, bb.reshape(1, 1),
                         Wd1, bd1.reshape(1, h2dim), Wd2, bd2.reshape(1, p),
                         n, npad, d, ngr, h2dim, p)

    return bnd[:n, 0], dom


# speedup vs baseline: 1.6420x; 1.3471x over previous
import functools

import jax
import jax.numpy as jnp
from jax import lax
from jax.experimental import pallas as pl
from jax.experimental.pallas import tpu as pltpu
from jax.experimental.pallas import tpu_sc as plsc

_NC = 2
_NS = 16
_NW = _NC * _NS
_CHUNK = 128
_BR = 1024


def _tc_degree(dst_row, d):
    e = dst_row.shape[1]
    bl = 2560
    grid = e // bl
    shift = d.bit_length() - 1

    def body(dstr_ref, out_ref, acc_scr):
        i = pl.program_id(0)

        @pl.when(i == 0)
        def _init():
            acc_scr[...] = jnp.zeros_like(acc_scr)

        dbr = dstr_ref[...]
        sub_iota = lax.broadcasted_iota(jnp.int32, (d, 1), 0)
        ohu_t = ((dbr & (d - 1)) == sub_iota).astype(jnp.bfloat16)
        ohv_t = (lax.shift_right_logical(dbr, shift) == sub_iota
                 ).astype(jnp.bfloat16)
        dn = (((1,), (1,)), ((), ()))
        acc_scr[...] += lax.dot_general(ohu_t, ohv_t, dn,
                                        preferred_element_type=jnp.float32)

        @pl.when(i == grid - 1)
        def _fin():
            out_ref[...] = acc_scr[...]

    return pl.pallas_call(
        body,
        grid=(grid,),
        in_specs=[pl.BlockSpec((1, bl), lambda i: (0, i))],
        out_specs=pl.BlockSpec((d, d), lambda i: (0, 0)),
        out_shape=jax.ShapeDtypeStruct((d, d), jnp.float32),
        scratch_shapes=[pltpu.VMEM((d, d), jnp.float32)],
    )(dst_row)


def _sc_edge_pass(xp, src2, dst2, npad, d, k0, k1):
    chunk = src2.shape[2]
    kmax = max(k0, k1)
    rps = npad // _NS
    mesh = plsc.VectorSubcoreMesh(core_axis_name="c", subcore_axis_name="s")

    @functools.partial(
        pl.kernel,
        mesh=mesh,
        out_type=jax.ShapeDtypeStruct((_NC, npad, d), jnp.float32),
        scratch_types=[
            pltpu.VMEM((2, 1, chunk), jnp.int32),
            pltpu.VMEM((kmax, 1, chunk), jnp.int32),
            pltpu.VMEM((2, chunk, d), jnp.float32),
            pltpu.VMEM_SHARED((npad, d), jnp.float32),
            pltpu.SemaphoreType.DMA((2,)),
            pltpu.SemaphoreType.DMA((2,)),
            pltpu.SemaphoreType.DMA((2,)),
        ],
    )
    def edge_kernel(xp_hbm, src_hbm, dst_hbm, out_hbm,
                    sidx_v, dst_v, rows_v, acc_sh, sem_g, sem_s, sem_w):
        cid = lax.axis_index("c")
        sid = lax.axis_index("s")
        kc = jnp.where(cid == 0, jnp.int32(k0), jnp.int32(k1))
        off = jnp.where(cid == 0, sid * k0, _NS * k0 + sid * k1)
        base = sid * rps

        z16 = jnp.zeros((16,), jnp.float32)

        def zrow(r, c):
            for l in range(d // 16):
                rows_v[0, r, pl.ds(l * 16, 16)] = z16
            return c

        lax.fori_loop(0, chunk, zrow, 0)

        def zcp(t, c):
            pltpu.sync_copy(rows_v.at[0],
                            acc_sh.at[pl.ds(base + t * chunk, chunk)])
            return c

        lax.fori_loop(0, rps // chunk, zcp, 0)
        pltpu.sync_copy(dst_hbm.at[pl.ds(off, kmax)], dst_v)
        plsc.subcore_barrier()

        pltpu.sync_copy(src_hbm.at[off], sidx_v.at[0])
        pltpu.async_copy(xp_hbm.at[sidx_v.at[0, 0]], rows_v.at[0], sem_g.at[0])
        pltpu.async_copy(src_hbm.at[off + 1], sidx_v.at[1], sem_s.at[1])

        def body(j, c):
            p = lax.rem(j, 2)
            q = lax.rem(j + 1, 2)

            @pl.when(j + 1 < kc)
            def _next_gather():
                @pl.when(j >= 1)
                def _drain_prev_scatter():
                    pltpu.make_async_copy(
                        rows_v.at[q], acc_sh.at[dst_v.at[j - 1, 0]],
                        sem_w.at[q]).wait()

                pltpu.make_async_copy(src_hbm.at[off + j + 1], sidx_v.at[q],
                                      sem_s.at[q]).wait()
                pltpu.async_copy(xp_hbm.at[sidx_v.at[q, 0]], rows_v.at[q],
                                 sem_g.at[q])

            pltpu.make_async_copy(xp_hbm.at[sidx_v.at[p, 0]], rows_v.at[p],
                                  sem_g.at[p]).wait()

            @pl.when(j + 2 < kc)
            def _next_sidx():
                pltpu.async_copy(src_hbm.at[off + j + 2], sidx_v.at[p],
                                 sem_s.at[p])

            pltpu.async_copy(rows_v.at[p], acc_sh.at[dst_v.at[j, 0]],
                             sem_w.at[p], add=True)
            return c

        lax.fori_loop(0, kc, body, 0)
        pltpu.make_async_copy(rows_v.at[lax.rem(kc, 2)],
                              acc_sh.at[dst_v.at[kc - 2, 0]],
                              sem_w.at[lax.rem(kc, 2)]).wait()
        pltpu.make_async_copy(rows_v.at[lax.rem(kc + 1, 2)],
                              acc_sh.at[dst_v.at[kc - 1, 0]],
                              sem_w.at[lax.rem(kc + 1, 2)]).wait()
        plsc.subcore_barrier()
        pltpu.sync_copy(acc_sh.at[pl.ds(base, rps)],
                        out_hbm.at[cid, pl.ds(base, rps)])

    return edge_kernel(xp, src2, dst2)


def _tc_scale(degcol, xpad, npad, d):
    grid = npad // _BR

    def body(deg_ref, x_ref, xp_ref, dinv_ref):
        dinv = 1.0 / jnp.sqrt(deg_ref[...] + 1.0)
        dinv_ref[...] = dinv
        xp_ref[...] = x_ref[...] * dinv

    return pl.pallas_call(
        body,
        grid=(grid,),
        in_specs=[
            pl.BlockSpec((_BR, 1), lambda i: (i, 0)),
            pl.BlockSpec((_BR, d), lambda i: (i, 0)),
        ],
        out_specs=[
            pl.BlockSpec((_BR, d), lambda i: (i, 0)),
            pl.BlockSpec((_BR, 1), lambda i: (i, 0)),
        ],
        out_shape=[
            jax.ShapeDtypeStruct((npad, d), jnp.float32),
            jax.ShapeDtypeStruct((npad, 1), jnp.float32),
        ],
    )(degcol, xpad)


def _elu(z):
    return jnp.where(z > 0, z, jnp.exp(z) - 1.0)


def _tc_conv_tail(acc2, xp, dinv, xres, w, b, g, be, n, npad, d):
    grid = npad // _BR

    def body(acc_ref, xp_ref, dinv_ref, x_ref, w_ref, b_ref, g_ref, be_ref,
             h_ref, xp2_ref):
        i = pl.program_id(0)
        s = acc_ref[0] + acc_ref[1] + xp_ref[...]
        agg = s * dinv_ref[...]
        pre = jnp.dot(agg, w_ref[...], preferred_element_type=jnp.float32,
                      precision=lax.Precision.HIGHEST) + b_ref[...]
        m = jnp.mean(pre, axis=-1, keepdims=True)
        v = jnp.mean((pre - m) ** 2, axis=-1, keepdims=True)
        ln = (pre - m) / jnp.sqrt(v + 1e-5) * g_ref[...] + be_ref[...]
        h = _elu(ln) + x_ref[...]
        h_ref[...] = h
        rows = i * _BR + lax.broadcasted_iota(jnp.int32, (_BR, 1), 0)
        mask = (rows < n).astype(jnp.float32)
        xp2_ref[...] = h * dinv_ref[...] * mask

    return pl.pallas_call(
        body,
        grid=(grid,),
        in_specs=[
            pl.BlockSpec((_NC, _BR, d), lambda i: (0, i, 0)),
            pl.BlockSpec((_BR, d), lambda i: (i, 0)),
            pl.BlockSpec((_BR, 1), lambda i: (i, 0)),
            pl.BlockSpec((_BR, d), lambda i: (i, 0)),
            pl.BlockSpec((d, d), lambda i: (0, 0)),
            pl.BlockSpec((1, d), lambda i: (0, 0)),
            pl.BlockSpec((1, d), lambda i: (0, 0)),
            pl.BlockSpec((1, d), lambda i: (0, 0)),
        ],
        out_specs=[
            pl.BlockSpec((_BR, d), lambda i: (i, 0)),
            pl.BlockSpec((_BR, d), lambda i: (i, 0)),
        ],
        out_shape=[
            jax.ShapeDtypeStruct((npad, d), jnp.float32),
            jax.ShapeDtypeStruct((npad, d), jnp.float32),
        ],
    )(acc2, xp, dinv, xres, w, b, g, be)


def _tc_final(acc2, xp2, dinv, hres, batch2d, w, b, g, be,
              wb, bb, wd1, bd1, wd2, bd2, n, npad, d, ngr, h2dim, p):
    grid = npad // _BR

    def body(acc_ref, xp_ref, dinv_ref, h_ref, bt_ref, w_ref, b_ref, g_ref,
             be_ref, wb_ref, bb_ref, wd1_ref, bd1_ref, wd2_ref, bd2_ref,
             bnd_ref, dom_ref, pool_scr, cnt_scr):
        i = pl.program_id(0)
        s = acc_ref[0] + acc_ref[1] + xp_ref[...]
        agg = s * dinv_ref[...]
        pre = jnp.dot(agg, w_ref[...], preferred_element_type=jnp.float32,
                      precision=lax.Precision.HIGHEST) + b_ref[...]
        m = jnp.mean(pre, axis=-1, keepdims=True)
        v = jnp.mean((pre - m) ** 2, axis=-1, keepdims=True)
        ln = (pre - m) / jnp.sqrt(v + 1e-5) * g_ref[...] + be_ref[...]
        h2 = _elu(ln) + h_ref[...]

        bnd_ref[...] = jnp.dot(h2, wb_ref[...],
                               preferred_element_type=jnp.float32,
                               precision=lax.Precision.HIGHEST) + bb_ref[...]

        @pl.when(i == 0)
        def _init():
            pool_scr[...] = jnp.zeros_like(pool_scr)
            cnt_scr[...] = jnp.zeros_like(cnt_scr)

        oh = (bt_ref[...] == lax.broadcasted_iota(jnp.int32, (1, ngr), 1)
              ).astype(jnp.float32)
        dn = (((0,), (0,)), ((), ()))
        pool_scr[...] += lax.dot_general(oh, h2, dn,
                                         preferred_element_type=jnp.float32,
                                         precision=lax.Precision.HIGHEST)
        cnt_scr[...] += lax.dot_general(oh, jnp.ones_like(h2), dn,
                                        preferred_element_type=jnp.float32,
                                        precision=lax.Precision.HIGHEST)

        @pl.when(i == grid - 1)
        def _fin():
            mean = pool_scr[...] / jnp.maximum(cnt_scr[...], 1.0)
            d1 = _elu(jnp.dot(mean, wd1_ref[...],
                              preferred_element_type=jnp.float32,
                              precision=lax.Precision.HIGHEST) + bd1_ref[...])
            dom_ref[...] = jnp.dot(d1, wd2_ref[...],
                                   preferred_element_type=jnp.float32,
                                   precision=lax.Precision.HIGHEST) + bd2_ref[...]

    return pl.pallas_call(
        body,
        grid=(grid,),
        in_specs=[
            pl.BlockSpec((_NC, _BR, d), lambda i: (0, i, 0)),
            pl.BlockSpec((_BR, d), lambda i: (i, 0)),
            pl.BlockSpec((_BR, 1), lambda i: (i, 0)),
            pl.BlockSpec((_BR, d), lambda i: (i, 0)),
            pl.BlockSpec((_BR, 1), lambda i: (i, 0)),
            pl.BlockSpec((d, d), lambda i: (0, 0)),
            pl.BlockSpec((1, d), lambda i: (0, 0)),
            pl.BlockSpec((1, d), lambda i: (0, 0)),
            pl.BlockSpec((1, d), lambda i: (0, 0)),
            pl.BlockSpec((d, 1), lambda i: (0, 0)),
            pl.BlockSpec((1, 1), lambda i: (0, 0)),
            pl.BlockSpec((d, h2dim), lambda i: (0, 0)),
            pl.BlockSpec((1, h2dim), lambda i: (0, 0)),
            pl.BlockSpec((h2dim, p), lambda i: (0, 0)),
            pl.BlockSpec((1, p), lambda i: (0, 0)),
        ],
        out_specs=[
            pl.BlockSpec((_BR, 1), lambda i: (i, 0)),
            pl.BlockSpec((ngr, p), lambda i: (0, 0)),
        ],
        out_shape=[
            jax.ShapeDtypeStruct((npad, 1), jnp.float32),
            jax.ShapeDtypeStruct((ngr, p), jnp.float32),
        ],
        scratch_shapes=[
            pltpu.VMEM((ngr, d), jnp.float32),
            pltpu.VMEM((ngr, d), jnp.float32),
        ],
    )(acc2, xp2, dinv, hres, batch2d, w, b, g, be,
      wb, bb, wd1, bd1, wd2, bd2)


def kernel(x, edge_index, batch, Wc1, bc1, g1, be1, Wc2, bc2, g2, be2,
           Wb, bb, Wd1, bd1, Wd2, bd2):
    n, d = x.shape
    e = edge_index.shape[1]
    ngr = 16
    h2dim = Wd1.shape[1]
    p = Wd2.shape[1]

    npad = -(-n // 2048) * 2048
    dummy = npad - 1

    f0 = 0.63
    c_total = -(-e // _CHUNK)
    k0 = max(2, -(-int(c_total * f0) // _NS))
    k1 = max(2, -(-(c_total - _NS * k0) // _NS))
    rows = _NS * (k0 + k1) + abs(k0 - k1)
    rows = -(-rows // 20) * 20

    src = edge_index[0].astype(jnp.int32)
    dst = edge_index[1].astype(jnp.int32)
    padi = jnp.full((rows * _CHUNK - e,), dummy, jnp.int32)
    src2 = jnp.concatenate([src, padi]).reshape(rows, 1, _CHUNK)
    dst2 = jnp.concatenate([dst, padi]).reshape(rows, 1, _CHUNK)

    xpad = jnp.pad(x, ((0, npad - n), (0, 0)))
    batch2d = jnp.pad(batch.astype(jnp.int32), (0, npad - n),
                      constant_values=ngr).reshape(npad, 1)

    degmat = _tc_degree(dst2.reshape(1, rows * _CHUNK), d)
    degcol = degmat.T.reshape(d * d, 1)[:npad]
    xp1, dinv = _tc_scale(degcol, xpad, npad, d)

    acc1 = _sc_edge_pass(xp1, src2, dst2, npad, d, k0, k1)
    h, xp2 = _tc_conv_tail(acc1, xp1, dinv, xpad, Wc1,
                           bc1.reshape(1, d), g1.reshape(1, d),
                           be1.reshape(1, d), n, npad, d)

    acc2 = _sc_edge_pass(xp2, src2, dst2, npad, d, k0, k1)
    bnd, dom = _tc_final(acc2, xp2, dinv, h, batch2d, Wc2,
                         bc2.reshape(1, d), g2.reshape(1, d),
                         be2.reshape(1, d), Wb, bb.reshape(1, 1),
                         Wd1, bd1.reshape(1, h2dim), Wd2, bd2.reshape(1, p),
                         n, npad, d, ngr, h2dim, p)

    return bnd[:n, 0], dom
